# Initial kernel scaffold; baseline (speedup 1.0000x reference)
import functools

import jax
import jax.numpy as jnp
from jax import lax
from jax.experimental import pallas as pl
from jax.experimental.pallas import tpu as pltpu
from jax.experimental.pallas import tpu_sc as plsc

NC = 2
NS = 16
NW = NC * NS

C = 128


@functools.partial(jax.jit, static_argnames=("n_chunks",))
def _sc_gather(W, idx, n_chunks):
    btot = idx.shape[0]
    d = W.shape[1]
    b_per_w = btot // NW

    mesh = plsc.VectorSubcoreMesh(core_axis_name="c", subcore_axis_name="s")

    @functools.partial(
        pl.kernel,
        out_type=jax.ShapeDtypeStruct((btot, d), jnp.float32),
        mesh=mesh,
        scratch_types=[
            pltpu.VMEM((C,), jnp.int32),
            pltpu.VMEM((C, d), jnp.float32),
            pltpu.SemaphoreType.DMA,
        ],
    )
    def body(table_hbm, idx_hbm, out_hbm, idx_v, rows_v, sem):
        wid = lax.axis_index("s") * NC + lax.axis_index("c")
        base = wid * b_per_w

        def chunk(i, carry):
            off = base + i * C
            pltpu.sync_copy(idx_hbm.at[pl.ds(off, C)], idx_v)
            pltpu.async_copy(table_hbm.at[idx_v], rows_v, sem).wait()
            pltpu.sync_copy(rows_v, out_hbm.at[pl.ds(off, C)])
            return carry

        lax.fori_loop(0, n_chunks, chunk, 0)

    return body(W, idx)


def kernel(token_ids, W):
    b, h = token_ids.shape
    d = W.shape[1]
    idx = token_ids.reshape(-1).astype(jnp.int32)
    btot = idx.shape[0]
    n_chunks = btot // (NW * C)
    out = _sc_gather(W, idx, n_chunks)
    return out.reshape(b, h, d)


# speedup vs baseline: 1.5747x; 1.5747x over previous
import functools

import jax
import jax.numpy as jnp
from jax import lax
from jax.experimental import pallas as pl
from jax.experimental.pallas import tpu as pltpu
from jax.experimental.pallas import tpu_sc as plsc

NC = 2
NS = 16
NW = NC * NS

C = 128


@functools.partial(jax.jit, static_argnames=("n_chunks",))
def _sc_gather(W, idx, n_chunks):
    btot = idx.shape[0]
    d = W.shape[1]
    b_per_w = btot // NW

    mesh = plsc.VectorSubcoreMesh(core_axis_name="c", subcore_axis_name="s")

    @functools.partial(
        pl.kernel,
        out_type=jax.ShapeDtypeStruct((btot, d), jnp.float32),
        mesh=mesh,
        scratch_types=[
            pltpu.VMEM((C,), jnp.int32),
            pltpu.VMEM((C, d), jnp.float32),
            pltpu.SemaphoreType.DMA,
        ],
        compiler_params=pltpu.CompilerParams(use_tc_tiling_on_sc=False),
    )
    def body(table_hbm, idx_hbm, out_hbm, idx_v, rows_v, sem):
        wid = lax.axis_index("s") * NC + lax.axis_index("c")
        base = wid * b_per_w

        def chunk(i, carry):
            off = base + i * C
            pltpu.sync_copy(idx_hbm.at[pl.ds(off, C)], idx_v)
            pltpu.async_copy(table_hbm.at[idx_v], rows_v, sem).wait()
            pltpu.sync_copy(rows_v, out_hbm.at[pl.ds(off, C)])
            return carry

        lax.fori_loop(0, n_chunks, chunk, 0)

    return body(W, idx)


def kernel(token_ids, W):
    b, h = token_ids.shape
    d = W.shape[1]
    idx = token_ids.reshape(-1).astype(jnp.int32)
    btot = idx.shape[0]
    n_chunks = btot // (NW * C)
    out = _sc_gather(W, idx, n_chunks)
    return out.reshape(b, h, d)


# speedup vs baseline: 1.7978x; 1.1417x over previous
import functools

import jax
import jax.numpy as jnp
from jax import lax
from jax.experimental import pallas as pl
from jax.experimental.pallas import tpu as pltpu
from jax.experimental.pallas import tpu_sc as plsc

NC = 2
NS = 16
NW = NC * NS

C = 128
K = 4


@functools.partial(jax.jit, static_argnames=("n_groups",))
def _sc_gather(W, idx2d, n_groups):
    n_rows, c = idx2d.shape
    btot = n_rows * c
    d = W.shape[1]
    rows_per_w = n_rows // NW

    mesh = plsc.VectorSubcoreMesh(core_axis_name="c", subcore_axis_name="s")

    @functools.partial(
        pl.kernel,
        out_type=jax.ShapeDtypeStruct((btot, d), jnp.float32),
        mesh=mesh,
        scratch_types=[
            pltpu.VMEM((K, C), jnp.int32),
            pltpu.VMEM((K, C, d), jnp.float32),
            pltpu.SemaphoreType.DMA,
            pltpu.SemaphoreType.DMA,
        ],
        compiler_params=pltpu.CompilerParams(use_tc_tiling_on_sc=False),
    )
    def body(table_hbm, idx_hbm, out_hbm, idx_v, rows_v, gsem, osem):
        wid = lax.axis_index("s") * NC + lax.axis_index("c")
        row_base = wid * rows_per_w

        def group(g, carry):
            roff = row_base + g * K
            pltpu.sync_copy(idx_hbm.at[pl.ds(roff, K)], idx_v)
            for b in range(K):
                pltpu.make_async_copy(
                    table_hbm.at[idx_v.at[b]], rows_v.at[b], gsem
                ).start()
            for b in range(K):
                pltpu.make_async_copy(
                    table_hbm.at[idx_v.at[b]], rows_v.at[b], gsem
                ).wait()
            for b in range(K):
                pltpu.make_async_copy(
                    rows_v.at[b], out_hbm.at[pl.ds((roff + b) * C, C)], osem
                ).start()
            for b in range(K):
                pltpu.make_async_copy(
                    rows_v.at[b], out_hbm.at[pl.ds((roff + b) * C, C)], osem
                ).wait()
            return carry

        lax.fori_loop(0, n_groups, group, 0)

    return body(W, idx2d)


def kernel(token_ids, W):
    b, h = token_ids.shape
    d = W.shape[1]
    idx = token_ids.reshape(-1).astype(jnp.int32)
    btot = idx.shape[0]
    idx2d = idx.reshape(btot // C, C)
    n_groups = btot // (NW * C * K)
    out = _sc_gather(W, idx2d, n_groups)
    return out.reshape(b, h, d)


# speedup vs baseline: 1.8532x; 1.0308x over previous
import functools

import jax
import jax.numpy as jnp
from jax import lax
from jax.experimental import pallas as pl
from jax.experimental.pallas import tpu as pltpu
from jax.experimental.pallas import tpu_sc as plsc

NC = 2
NS = 16
NW = NC * NS

C = 128
K = 8


@functools.partial(jax.jit, static_argnames=("n_groups",))
def _sc_gather(W, idx2d, n_groups):
    n_rows, c = idx2d.shape
    btot = n_rows * c
    d = W.shape[1]
    rows_per_w = n_rows // NW

    mesh = plsc.VectorSubcoreMesh(core_axis_name="c", subcore_axis_name="s")

    @functools.partial(
        pl.kernel,
        out_type=jax.ShapeDtypeStruct((btot, d), jnp.float32),
        mesh=mesh,
        scratch_types=[
            pltpu.VMEM((K, C), jnp.int32),
            pltpu.VMEM((K, C, d), jnp.float32),
            pltpu.SemaphoreType.DMA((K,)),
            pltpu.SemaphoreType.DMA((K,)),
        ],
        compiler_params=pltpu.CompilerParams(use_tc_tiling_on_sc=False),
    )
    def body(table_hbm, idx_hbm, out_hbm, idx_v, rows_v, gsem, osem):
        wid = lax.axis_index("s") * NC + lax.axis_index("c")
        row_base = wid * rows_per_w

        def group(g, carry):
            roff = row_base + g * K
            pltpu.sync_copy(idx_hbm.at[pl.ds(roff, K)], idx_v)
            for b in range(K):
                pltpu.make_async_copy(
                    table_hbm.at[idx_v.at[b]], rows_v.at[b], gsem.at[b]
                ).start()
            for b in range(K):
                pltpu.make_async_copy(
                    table_hbm.at[idx_v.at[b]], rows_v.at[b], gsem.at[b]
                ).wait()
                pltpu.make_async_copy(
                    rows_v.at[b], out_hbm.at[pl.ds((roff + b) * C, C)], osem.at[b]
                ).start()
            for b in range(K):
                pltpu.make_async_copy(
                    rows_v.at[b], out_hbm.at[pl.ds((roff + b) * C, C)], osem.at[b]
                ).wait()
            return carry

        lax.fori_loop(0, n_groups, group, 0)

    return body(W, idx2d)


def kernel(token_ids, W):
    b, h = token_ids.shape
    d = W.shape[1]
    idx = token_ids.reshape(-1).astype(jnp.int32)
    btot = idx.shape[0]
    idx2d = idx.reshape(btot // C, C)
    n_groups = btot // (NW * C * K)
    out = _sc_gather(W, idx2d, n_groups)
    return out.reshape(b, h, d)
